# baseline (device time: 68953 ns/iter reference)
import jax
import jax.numpy as jnp
from jax import lax
from jax.experimental import pallas as pl
from jax.experimental.pallas import tpu as pltpu

N_DEV = 8
LOG = 3

_ORDERS = ("xyz", "yzx", "zxy", "xyz", "yzx", "zyx", "xyz", "zxy") * 2
PARTS = tuple(
    (i * 128, 128, o) for i, o in enumerate(_ORDERS)
)


def kernel(x, w_mat):
    m, _ = x.shape
    n = w_mat.shape[1]
    np_ = len(PARTS)

    def body(x_hbm, w_hbm, out_hbm, *sc):
        xv32, wv32, xv, wv = sc[0:4]
        idx = 4
        bufs = []
        for _ in PARTS:
            bufs.append(dict(sb=sc[idx:idx + 3], rv=sc[idx + 3:idx + 6],
                             ag=sc[idx + 6]))
            idx += 7
        ssems, rsems, in_sems, io_sems = sc[idx:idx + 4]

        p = lax.axis_index("i")
        b0 = p & 1
        b1 = (p >> 1) & 1
        b2 = (p >> 2) & 1
        bit = {"x": b0 ^ b1, "y": b1, "z": b2}
        partner = {"x": p ^ 1, "y": p ^ 3, "z": p ^ 4}

        ld_x = pltpu.make_async_copy(x_hbm, xv32, in_sems.at[0])
        ld_w = pltpu.make_async_copy(w_hbm, wv32, in_sems.at[1])
        ld_x.start()
        ld_w.start()

        barrier_sem = pltpu.get_barrier_semaphore()
        for d in "xyz":
            pl.semaphore_signal(
                barrier_sem, inc=1,
                device_id=(partner[d],), device_id_type=pl.DeviceIdType.MESH,
            )
        pl.semaphore_wait(barrier_sem, 3)
        ld_x.wait()
        ld_w.wait()
        xv[...] = xv32[...].astype(jnp.bfloat16)
        wv[...] = wv32[...].astype(jnp.bfloat16)

        parts = []
        for i, (clo, cw, order) in enumerate(PARTS):
            parts.append(dict(
                cs=slice(clo, clo + cw), order=order, bufs=bufs[i],
                lo=[0 * p], rs=[], ag=[], i=i,
            ))

        def exch(i, src, dst, sidx, dev):
            rd = pltpu.make_async_remote_copy(
                src_ref=src, dst_ref=dst,
                send_sem=ssems.at[i, sidx], recv_sem=rsems.at[i, sidx],
                device_id=(dev,), device_id_type=pl.DeviceIdType.MESH,
            )
            rd.start()
            return rd

        for k in range(LOG):
            half = m >> (k + 1)
            vals = {}
            for pt in parts:
                b = bit[pt["order"][k]]
                send_lo = pt["lo"][k] + (1 - b) * half
                vals[pt["i"]] = jnp.dot(
                    xv[pl.ds(send_lo, half), :], wv[:, pt["cs"]],
                    preferred_element_type=jnp.float32,
                )
                pt["send_lo"] = send_lo
            for pt in parts:
                i = pt["i"]
                d = pt["order"][k]
                b = bit[d]
                if k > 0:
                    pt["rs"][k - 1].wait_recv()
                val = vals[i]
                for j in range(k):
                    val = val + pt["bufs"]["rv"][j][
                        pl.ds(pt["send_lo"] - pt["lo"][j + 1], half), :
                    ].astype(jnp.float32)
                pt["bufs"]["sb"][k][...] = val.astype(jnp.bfloat16)
                pt["rs"].append(exch(
                    i, pt["bufs"]["sb"][k], pt["bufs"]["rv"][k], k, partner[d]
                ))
                pt["lo"].append(pt["lo"][k] + b * half)

        own = m >> LOG
        io_cp = []
        for pt in parts:
            i = pt["i"]
            pt["rs"][LOG - 1].wait_recv()
            olo = pt["lo"][LOG]
            fin = jnp.dot(
                xv[pl.ds(olo, own), :], wv[:, pt["cs"]],
                preferred_element_type=jnp.float32,
            )
            for j in range(LOG):
                fin = fin + pt["bufs"]["rv"][j][
                    pl.ds(olo - pt["lo"][j + 1], own), :
                ].astype(jnp.float32)
            agb = pt["bufs"]["ag"]
            agb[pl.ds(olo, own), :] = fin.astype(jnp.bfloat16)
            d = pt["order"][LOG - 1]
            pt["ag"].append(exch(
                i, agb.at[pl.ds(olo, own)], agb.at[pl.ds(olo, own)],
                LOG, partner[d],
            ))
            cp = pltpu.make_async_copy(
                agb.at[pl.ds(olo, own)],
                out_hbm.at[pl.ds(olo, own), pt["cs"]],
                io_sems.at[i, 0],
            )
            cp.start()
            io_cp.append(cp)
            pt["blo"] = olo

        for t in range(LOG):
            sz = own << t
            for pt in parts:
                i = pt["i"]
                d = pt["order"][LOG - 1 - t]
                b = bit[d]
                pt["ag"][t].wait_recv()
                plo = pt["blo"] + (1 - 2 * b) * sz
                merged_lo = pt["blo"] - b * sz
                agb = pt["bufs"]["ag"]
                if t < LOG - 1:
                    d_next = pt["order"][LOG - 2 - t]
                    pt["ag"].append(exch(
                        i, agb.at[pl.ds(merged_lo, 2 * sz)],
                        agb.at[pl.ds(merged_lo, 2 * sz)],
                        LOG + 1 + t, partner[d_next],
                    ))
                cp = pltpu.make_async_copy(
                    agb.at[pl.ds(plo, sz)],
                    out_hbm.at[pl.ds(plo, sz), pt["cs"]],
                    io_sems.at[i, 1 + t],
                )
                cp.start()
                io_cp.append(cp)
                pt["blo"] = merged_lo

        for cp in io_cp:
            cp.wait()
        for pt in parts:
            for rd in pt["rs"]:
                rd.wait_send()
            for rd in pt["ag"]:
                rd.wait_send()

    scratch = [
        pltpu.VMEM((m, x.shape[1]), jnp.float32),
        pltpu.VMEM((w_mat.shape[0], n), jnp.float32),
        pltpu.VMEM((m, x.shape[1]), jnp.bfloat16),
        pltpu.VMEM((w_mat.shape[0], n), jnp.bfloat16),
    ]
    for _, cw, _ in PARTS:
        for k in range(LOG):
            scratch.append(pltpu.VMEM((m >> (k + 1), cw), jnp.bfloat16))
        for k in range(LOG):
            scratch.append(pltpu.VMEM((m >> (k + 1), cw), jnp.bfloat16))
        scratch.append(pltpu.VMEM((m, cw), jnp.bfloat16))
    scratch += [
        pltpu.SemaphoreType.DMA((np_, 2 * LOG)),
        pltpu.SemaphoreType.DMA((np_, 2 * LOG)),
        pltpu.SemaphoreType.DMA((2,)),
        pltpu.SemaphoreType.DMA((np_, LOG + 1)),
    ]

    return pl.pallas_call(
        body,
        out_shape=jax.ShapeDtypeStruct((m, n), jnp.bfloat16),
        in_specs=[
            pl.BlockSpec(memory_space=pltpu.MemorySpace.HBM),
            pl.BlockSpec(memory_space=pltpu.MemorySpace.HBM),
        ],
        out_specs=pl.BlockSpec(memory_space=pltpu.MemorySpace.HBM),
        scratch_shapes=scratch,
        compiler_params=pltpu.CompilerParams(collective_id=0),
    )(x, w_mat)


# device time: 68627 ns/iter; 1.0048x vs baseline; 1.0048x over previous
import jax
import jax.numpy as jnp
from jax import lax
from jax.experimental import pallas as pl
from jax.experimental.pallas import tpu as pltpu

N_DEV = 8
LOG = 3

_ORDERS = ("xyz", "yzx", "zxy", "xyz", "yzx", "zyx", "xyz", "zxy")
PARTS = tuple(
    (i * 256, 256, o) for i, o in enumerate(_ORDERS)
)


def kernel(x, w_mat):
    m, _ = x.shape
    n = w_mat.shape[1]
    np_ = len(PARTS)

    def body(x_hbm, w_hbm, out_hbm, *sc):
        xv32, wv32, xv, wv = sc[0:4]
        idx = 4
        bufs = []
        for _ in PARTS:
            bufs.append(dict(sb=sc[idx:idx + 3], rv=sc[idx + 3:idx + 6],
                             ag=sc[idx + 6]))
            idx += 7
        ssems, rsems, in_sems, io_sems = sc[idx:idx + 4]

        p = lax.axis_index("i")
        b0 = p & 1
        b1 = (p >> 1) & 1
        b2 = (p >> 2) & 1
        bit = {"x": b0 ^ b1, "y": b1, "z": b2}
        partner = {"x": p ^ 1, "y": p ^ 3, "z": p ^ 4}

        ld_x = pltpu.make_async_copy(x_hbm, xv32, in_sems.at[0])
        ld_w = pltpu.make_async_copy(w_hbm, wv32, in_sems.at[1])
        ld_x.start()
        ld_w.start()

        barrier_sem = pltpu.get_barrier_semaphore()
        for d in "xyz":
            pl.semaphore_signal(
                barrier_sem, inc=1,
                device_id=(partner[d],), device_id_type=pl.DeviceIdType.MESH,
            )
        pl.semaphore_wait(barrier_sem, 3)
        ld_x.wait()
        ld_w.wait()
        xv[...] = xv32[...].astype(jnp.bfloat16)
        wv[...] = wv32[...].astype(jnp.bfloat16)

        parts = []
        for i, (clo, cw, order) in enumerate(PARTS):
            parts.append(dict(
                cs=slice(clo, clo + cw), order=order, bufs=bufs[i],
                lo=[0 * p], rs=[], ag=[], i=i,
            ))

        def exch(i, src, dst, sidx, dev):
            rd = pltpu.make_async_remote_copy(
                src_ref=src, dst_ref=dst,
                send_sem=ssems.at[i, sidx], recv_sem=rsems.at[i, sidx],
                device_id=(dev,), device_id_type=pl.DeviceIdType.MESH,
            )
            rd.start()
            return rd

        for k in range(LOG):
            half = m >> (k + 1)
            vals = {}
            for pt in parts:
                b = bit[pt["order"][k]]
                send_lo = pt["lo"][k] + (1 - b) * half
                vals[pt["i"]] = jnp.dot(
                    xv[pl.ds(send_lo, half), :], wv[:, pt["cs"]],
                    preferred_element_type=jnp.float32,
                )
                pt["send_lo"] = send_lo
            for pt in parts:
                i = pt["i"]
                d = pt["order"][k]
                b = bit[d]
                if k > 0:
                    pt["rs"][k - 1].wait_recv()
                val = vals[i]
                for j in range(k):
                    val = val + pt["bufs"]["rv"][j][
                        pl.ds(pt["send_lo"] - pt["lo"][j + 1], half), :
                    ].astype(jnp.float32)
                pt["bufs"]["sb"][k][...] = val.astype(jnp.bfloat16)
                pt["rs"].append(exch(
                    i, pt["bufs"]["sb"][k], pt["bufs"]["rv"][k], k, partner[d]
                ))
                pt["lo"].append(pt["lo"][k] + b * half)

        own = m >> LOG
        io_cp = []
        for pt in parts:
            i = pt["i"]
            pt["rs"][LOG - 1].wait_recv()
            olo = pt["lo"][LOG]
            fin = jnp.dot(
                xv[pl.ds(olo, own), :], wv[:, pt["cs"]],
                preferred_element_type=jnp.float32,
            )
            for j in range(LOG):
                fin = fin + pt["bufs"]["rv"][j][
                    pl.ds(olo - pt["lo"][j + 1], own), :
                ].astype(jnp.float32)
            agb = pt["bufs"]["ag"]
            agb[pl.ds(olo, own), :] = fin.astype(jnp.bfloat16)
            d = pt["order"][LOG - 1]
            pt["ag"].append(exch(
                i, agb.at[pl.ds(olo, own)], agb.at[pl.ds(olo, own)],
                LOG, partner[d],
            ))
            cp = pltpu.make_async_copy(
                agb.at[pl.ds(olo, own)],
                out_hbm.at[pl.ds(olo, own), pt["cs"]],
                io_sems.at[i, 0],
            )
            cp.start()
            io_cp.append(cp)
            pt["blo"] = olo

        for t in range(LOG):
            sz = own << t
            for pt in parts:
                i = pt["i"]
                d = pt["order"][LOG - 1 - t]
                b = bit[d]
                pt["ag"][t].wait_recv()
                plo = pt["blo"] + (1 - 2 * b) * sz
                merged_lo = pt["blo"] - b * sz
                agb = pt["bufs"]["ag"]
                if t < LOG - 1:
                    d_next = pt["order"][LOG - 2 - t]
                    pt["ag"].append(exch(
                        i, agb.at[pl.ds(merged_lo, 2 * sz)],
                        agb.at[pl.ds(merged_lo, 2 * sz)],
                        LOG + 1 + t, partner[d_next],
                    ))
                cp = pltpu.make_async_copy(
                    agb.at[pl.ds(plo, sz)],
                    out_hbm.at[pl.ds(plo, sz), pt["cs"]],
                    io_sems.at[i, 1 + t],
                )
                cp.start()
                io_cp.append(cp)
                pt["blo"] = merged_lo

        for cp in io_cp:
            cp.wait()
        for pt in parts:
            for rd in pt["rs"]:
                rd.wait_send()
            for rd in pt["ag"]:
                rd.wait_send()

    scratch = [
        pltpu.VMEM((m, x.shape[1]), jnp.float32),
        pltpu.VMEM((w_mat.shape[0], n), jnp.float32),
        pltpu.VMEM((m, x.shape[1]), jnp.bfloat16),
        pltpu.VMEM((w_mat.shape[0], n), jnp.bfloat16),
    ]
    for _, cw, _ in PARTS:
        for k in range(LOG):
            scratch.append(pltpu.VMEM((m >> (k + 1), cw), jnp.bfloat16))
        for k in range(LOG):
            scratch.append(pltpu.VMEM((m >> (k + 1), cw), jnp.bfloat16))
        scratch.append(pltpu.VMEM((m, cw), jnp.bfloat16))
    scratch += [
        pltpu.SemaphoreType.DMA((np_, 2 * LOG)),
        pltpu.SemaphoreType.DMA((np_, 2 * LOG)),
        pltpu.SemaphoreType.DMA((2,)),
        pltpu.SemaphoreType.DMA((np_, LOG + 1)),
    ]

    return pl.pallas_call(
        body,
        out_shape=jax.ShapeDtypeStruct((m, n), jnp.bfloat16),
        in_specs=[
            pl.BlockSpec(memory_space=pltpu.MemorySpace.HBM),
            pl.BlockSpec(memory_space=pltpu.MemorySpace.HBM),
        ],
        out_specs=pl.BlockSpec(memory_space=pltpu.MemorySpace.HBM),
        scratch_shapes=scratch,
        compiler_params=pltpu.CompilerParams(collective_id=0),
    )(x, w_mat)
